# baseline (speedup 1.0000x reference)
import functools

import jax
import jax.numpy as jnp
from jax import lax
from jax.experimental import pallas as pl
from jax.experimental.pallas import tpu as pltpu
from jax.experimental.pallas import tpu_sc as plsc

UPDATE_N = 100000
VOCAB_N = 1000000
D = 32
L = 16
NC, NS = 2, 16
NW = NC * NS
B_ROWS = 4096 * 200
ROWS_PER_W = B_ROWS // NW
CHUNK = 1024
N_CHUNKS = ROWS_PER_W // CHUNK

_mesh = plsc.VectorSubcoreMesh(core_axis_name="c", subcore_axis_name="s")


@functools.partial(
    pl.kernel,
    out_type=jax.ShapeDtypeStruct((B_ROWS, D), jnp.float32),
    mesh=_mesh,
    compiler_params=pltpu.CompilerParams(use_tc_tiling_on_sc=False),
    scratch_types=[
        pltpu.VMEM((CHUNK,), jnp.int32),
        pltpu.VMEM((CHUNK,), jnp.int32),
        pltpu.VMEM((CHUNK, D), jnp.float32),
        pltpu.VMEM((CHUNK, D), jnp.float32),
        pltpu.SemaphoreType.DMA,
        pltpu.SemaphoreType.DMA,
    ],
)
def _sc_lookup(idx_hbm, wu_hbm, wf_hbm, out_hbm, idxv, uidxv, ubuf, fbuf,
               sem_u, sem_f):
    wid = lax.axis_index("s") * NC + lax.axis_index("c")
    base = wid * ROWS_PER_W

    def chunk_body(ci, carry):
        start = base + ci * CHUNK
        pltpu.sync_copy(idx_hbm.at[pl.ds(start, CHUNK)], idxv)

        def clamp_body(j, carry2):
            v = idxv[pl.ds(j * L, L)]
            uidxv[pl.ds(j * L, L)] = jnp.minimum(v, UPDATE_N - 1)
            return carry2

        lax.fori_loop(0, CHUNK // L, clamp_body, 0)

        cu = pltpu.async_copy(wu_hbm.at[uidxv], ubuf, sem_u)
        cf = pltpu.async_copy(wf_hbm.at[idxv], fbuf, sem_f)
        cu.wait()
        cf.wait()

        def blend_body(g, carry2):
            vi = idxv[pl.ds(g * L, L)]
            for k in range(L):
                r = g * L + k
                m = vi[k] < UPDATE_N
                for h in range(D // L):
                    u = ubuf[r, pl.ds(h * L, L)]
                    f = fbuf[r, pl.ds(h * L, L)]
                    fbuf[r, pl.ds(h * L, L)] = jnp.where(m, u, f)
            return carry2

        lax.fori_loop(0, CHUNK // L, blend_body, 0)

        pltpu.sync_copy(fbuf, out_hbm.at[pl.ds(start, CHUNK)])
        return carry

    lax.fori_loop(0, N_CHUNKS, chunk_body, 0)


def kernel(inp, W_update, W_fixed):
    idx = inp.reshape(B_ROWS).astype(jnp.int32)
    out = _sc_lookup(idx, W_update, W_fixed)
    return out.reshape(inp.shape[0], inp.shape[1], D)
